# baseline (speedup 1.0000x reference)
import functools

import jax
import jax.numpy as jnp
from jax import lax
from jax.experimental import pallas as pl
from jax.experimental.pallas import tpu as pltpu
from jax.experimental.pallas import tpu_sc as plsc

V = 1000000
D = 64
B = 16384
K = 20
KP = K + 1

NC = 2
NS = 16
NW = NC * NS
BW = B // NW
CB = 16
NCHUNK = BW // CB
U_ROWS = CB * KP

_mesh = plsc.VectorSubcoreMesh(
    core_axis_name="c", subcore_axis_name="s", num_cores=NC, num_subcores=NS
)


@functools.partial(
    pl.kernel,
    out_type=(
        jax.ShapeDtypeStruct((B,), jnp.float32),
        jax.ShapeDtypeStruct((B * K,), jnp.float32),
    ),
    mesh=_mesh,
    compiler_params=pltpu.CompilerParams(
        needs_layout_passes=False, use_tc_tiling_on_sc=True),
    scratch_types=[
        pltpu.VMEM((BW,), jnp.int32),
        pltpu.VMEM((BW,), jnp.int32),
        pltpu.VMEM((BW * KP,), jnp.int32),
        pltpu.VMEM((BW * KP,), jnp.int32),
        pltpu.VMEM((2, CB, 128), jnp.float32),
        pltpu.VMEM((2, U_ROWS, 128), jnp.float32),
        pltpu.VMEM((BW,), jnp.float32),
        pltpu.VMEM((BW * K,), jnp.float32),
        pltpu.SemaphoreType.DMA,
        pltpu.SemaphoreType.DMA,
    ],
)
def _sc_scores(v_hbm, u_hbm, cpair_hbm, chalf_hbm, upair_hbm, uhalf_hbm,
               pos_out, neg_out,
               cpair, chalf, upair, uhalf, v_buf, u_buf, pos_sc, neg_sc,
               sem0, sem1):
    wid = lax.axis_index("s") * NC + lax.axis_index("c")

    pltpu.sync_copy(cpair_hbm.at[pl.ds(wid * BW, BW)], cpair)
    pltpu.sync_copy(chalf_hbm.at[pl.ds(wid * BW, BW)], chalf)
    pltpu.sync_copy(upair_hbm.at[pl.ds(wid * BW * KP, BW * KP)], upair)
    pltpu.sync_copy(uhalf_hbm.at[pl.ds(wid * BW * KP, BW * KP)], uhalf)

    iota = lax.iota(jnp.int32, 16)
    sems = (sem0, sem1)

    def copies(c, s):
        sem = sems[s]
        return [
            pltpu.make_async_copy(
                v_hbm.at[cpair.at[pl.ds(c * CB, CB)]],
                v_buf.at[s], sem),
            pltpu.make_async_copy(
                u_hbm.at[upair.at[pl.ds(c * U_ROWS, U_ROWS)]],
                u_buf.at[s], sem),
        ]

    def issue(c, s):
        for cp in copies(c, s):
            cp.start()

    def drain(c, s):
        for cp in copies(c, s):
            cp.wait()

    def compute(c, s):
        vb, ub = v_buf.at[s], u_buf.at[s]
        bvec = iota
        base = c * CB
        urow0 = bvec * KP
        cq = plsc.load_gather(chalf, [base + bvec])
        hq = [plsc.load_gather(uhalf, [base * KP + urow0 + j])
              for j in range(KP)]
        acc0 = (jnp.zeros((16,), jnp.float32),) * KP

        def dbody(d, accs):
            dvec = jnp.full((16,), d, jnp.int32)
            vv = plsc.load_gather(vb, [bvec, cq + dvec])
            pv = plsc.load_gather(ub, [urow0, hq[0] + dvec])
            out = [accs[0] + vv * pv]
            for k in range(K):
                nv = plsc.load_gather(ub, [urow0 + (k + 1), hq[k + 1] + dvec])
                out.append(accs[k + 1] + vv * nv)
            return tuple(out)

        accs = lax.fori_loop(0, D, dbody, acc0)
        pos_sc[pl.ds(base, 16)] = accs[0]
        nvec = (base + iota) * K
        for k in range(K):
            plsc.store_scatter(neg_sc, [nvec + k], accs[k + 1])

    issue(0, 0)
    issue(1, 1)

    def outer(i, carry):
        cc = i * 2
        for s in range(2):
            c = cc + s
            drain(c, s)
            compute(c, s)

            @pl.when(c + 2 < NCHUNK)
            def _():
                issue(c + 2, s)

        return carry

    lax.fori_loop(0, NCHUNK // 2, outer, 0)

    pltpu.sync_copy(pos_sc, pos_out.at[pl.ds(wid * BW, BW)])
    pltpu.sync_copy(neg_sc, neg_out.at[pl.ds(wid * BW * K, BW * K)])


def _loss_body(p_ref, n_ref, o_ref):
    def logsig(x):
        return jnp.minimum(x, 0.0) - jnp.log1p(jnp.exp(-jnp.abs(x)))

    tot = jnp.sum(logsig(p_ref[...])) + jnp.sum(logsig(-n_ref[...]))
    o_ref[0, 0] = -tot / jnp.float32(B)


_loss_call = pl.pallas_call(
    _loss_body,
    out_shape=jax.ShapeDtypeStruct((1, 1), jnp.float32),
    out_specs=pl.BlockSpec(memory_space=pltpu.MemorySpace.SMEM),
)


def kernel(center_words, pos_words, neg_words, v_embed, u_embed):
    v2 = v_embed.reshape(V // 2, 2 * D)
    u2 = u_embed.reshape(V // 2, 2 * D)
    iu = jnp.concatenate([pos_words[:, None], neg_words], axis=1).reshape(-1)
    cpair, chalf = center_words // 2, (center_words % 2) * D
    upair, uhalf = iu // 2, (iu % 2) * D
    pos_s, neg_s = _sc_scores(v2, u2, cpair, chalf, upair, uhalf)
    loss = _loss_call(pos_s.reshape(128, 128), neg_s.reshape(B * K // 128, 128))
    return loss[0, 0]


# speedup vs baseline: 1.0353x; 1.0353x over previous
import functools

import jax
import jax.numpy as jnp
from jax import lax
from jax.experimental import pallas as pl
from jax.experimental.pallas import tpu as pltpu
from jax.experimental.pallas import tpu_sc as plsc

V = 1000000
D = 64
B = 16384
K = 20
KP = K + 1

NC = 2
NS = 16
NW = NC * NS
BW = B // NW
CB = 16
NCHUNK = BW // CB
U_ROWS = CB * KP
NBUF = 4

_mesh = plsc.VectorSubcoreMesh(
    core_axis_name="c", subcore_axis_name="s", num_cores=NC, num_subcores=NS
)


@functools.partial(
    pl.kernel,
    out_type=(
        jax.ShapeDtypeStruct((B,), jnp.float32),
        jax.ShapeDtypeStruct((B, K), jnp.float32),
    ),
    mesh=_mesh,
    compiler_params=pltpu.CompilerParams(
        needs_layout_passes=False, use_tc_tiling_on_sc=False),
    scratch_types=[
        pltpu.VMEM((BW,), jnp.int32),
        pltpu.VMEM((BW * KP,), jnp.int32),
        pltpu.VMEM((NBUF, CB, D), jnp.float32),
        pltpu.VMEM((NBUF, U_ROWS, D), jnp.float32),
        pltpu.VMEM((BW,), jnp.float32),
        pltpu.VMEM((BW, K), jnp.float32),
        pltpu.SemaphoreType.DMA,
        pltpu.SemaphoreType.DMA,
        pltpu.SemaphoreType.DMA,
        pltpu.SemaphoreType.DMA,
    ],
)
def _sc_scores(v_hbm, u_hbm, cidx_hbm, uidx_hbm,
               pos_out, neg_out,
               cidx, uidx, v_buf, u_buf, pos_sc, neg_sc,
               sem0, sem1, sem2, sem3):
    wid = lax.axis_index("s") * NC + lax.axis_index("c")

    pltpu.sync_copy(cidx_hbm.at[pl.ds(wid * BW, BW)], cidx)
    pltpu.sync_copy(uidx_hbm.at[pl.ds(wid * BW * KP, BW * KP)], uidx)

    iota = lax.iota(jnp.int32, 16)
    sems = (sem0, sem1, sem2, sem3)

    def copies(c, s):
        sem = sems[s]
        return [
            pltpu.make_async_copy(
                v_hbm.at[cidx.at[pl.ds(c * CB, CB)]], v_buf.at[s], sem),
            pltpu.make_async_copy(
                u_hbm.at[uidx.at[pl.ds(c * U_ROWS, U_ROWS)]],
                u_buf.at[s], sem),
        ]

    def issue(c, s):
        for cp in copies(c, s):
            cp.start()

    def drain(c, s):
        for cp in copies(c, s):
            cp.wait()

    def compute(c, s):
        vb, ub = v_buf.at[s], u_buf.at[s]
        bvec = iota
        urow0 = bvec * KP
        acc0 = (jnp.zeros((16,), jnp.float32),) * KP

        def dbody(d, accs):
            dvec = jnp.full((16,), d, jnp.int32)
            vv = plsc.load_gather(vb, [bvec, dvec])
            pv = plsc.load_gather(ub, [urow0, dvec])
            out = [accs[0] + vv * pv]
            for k in range(K):
                nv = plsc.load_gather(ub, [urow0 + (k + 1), dvec])
                out.append(accs[k + 1] + vv * nv)
            return tuple(out)

        accs = lax.fori_loop(0, D, dbody, acc0)
        base = c * CB
        pos_sc[pl.ds(base, 16)] = accs[0]
        blvec = base + iota
        for k in range(K):
            plsc.store_scatter(
                neg_sc, [blvec, jnp.full((16,), k, jnp.int32)], accs[k + 1])

    for s in range(NBUF):
        issue(s, s)

    def outer(i, carry):
        cc = i * NBUF
        for s in range(NBUF):
            c = cc + s
            drain(c, s)
            compute(c, s)

            @pl.when(c + NBUF < NCHUNK)
            def _():
                issue(c + NBUF, s)

        return carry

    lax.fori_loop(0, NCHUNK // NBUF, outer, 0)

    pltpu.sync_copy(pos_sc, pos_out.at[pl.ds(wid * BW, BW)])
    pltpu.sync_copy(neg_sc, neg_out.at[pl.ds(wid * BW, BW)])


def _loss_body(p_ref, n_ref, o_ref):
    def logsig(x):
        return jnp.minimum(x, 0.0) - jnp.log1p(jnp.exp(-jnp.abs(x)))

    tot = jnp.sum(logsig(p_ref[...])) + jnp.sum(logsig(-n_ref[...]))
    o_ref[0, 0] = -tot / jnp.float32(B)


_loss_call = pl.pallas_call(
    _loss_body,
    out_shape=jax.ShapeDtypeStruct((1, 1), jnp.float32),
    out_specs=pl.BlockSpec(memory_space=pltpu.MemorySpace.SMEM),
)


def kernel(center_words, pos_words, neg_words, v_embed, u_embed):
    iu = jnp.concatenate([pos_words[:, None], neg_words], axis=1).reshape(-1)
    pos_s, neg_s = _sc_scores(v_embed, u_embed, center_words, iu)
    loss = _loss_call(pos_s.reshape(128, 128), neg_s.reshape(B * K // 128, 128))
    return loss[0, 0]


# speedup vs baseline: 1.0394x; 1.0040x over previous
import functools

import jax
import jax.numpy as jnp
from jax import lax
from jax.experimental import pallas as pl
from jax.experimental.pallas import tpu as pltpu
from jax.experimental.pallas import tpu_sc as plsc

V = 1000000
D = 64
B = 16384
K = 20

NC = 2
NS = 16
NW = NC * NS
BW = B // NW
CB = 32
NCHUNK = BW // CB
NEG_ROWS = CB * K

_mesh = plsc.VectorSubcoreMesh(
    core_axis_name="c", subcore_axis_name="s", num_cores=NC, num_subcores=NS
)

@functools.partial(
    pl.kernel,
    out_type=(
        jax.ShapeDtypeStruct((B,), jnp.float32),
        jax.ShapeDtypeStruct((B, K), jnp.float32),
    ),
    mesh=_mesh,
    compiler_params=pltpu.CompilerParams(
        needs_layout_passes=False, use_tc_tiling_on_sc=False),
    scratch_types=[
        pltpu.VMEM((NCHUNK, CB), jnp.int32),
        pltpu.VMEM((NCHUNK, CB), jnp.int32),
        pltpu.VMEM((BW * K // 128, 128), jnp.int32),
        pltpu.VMEM((2, CB, D), jnp.float32),
        pltpu.VMEM((2, CB, D), jnp.float32),
        pltpu.VMEM((2, NEG_ROWS, D), jnp.float32),
        pltpu.VMEM((BW,), jnp.float32),
        pltpu.VMEM((BW, K), jnp.float32),
        pltpu.SemaphoreType.DMA,
        pltpu.SemaphoreType.DMA,
    ],
)
def _sc_scores(v_hbm, u_hbm, cidx_hbm, pidx_hbm, nidx_hbm,
               pos_out, neg_out,
               cidx, pidx, nidx, v_buf, p_buf, n_buf, pos_sc, neg_sc,
               sem0, sem1):
    wid = lax.axis_index("s") * NC + lax.axis_index("c")

    pltpu.sync_copy(cidx_hbm.at[pl.ds(wid * NCHUNK, NCHUNK)], cidx)
    pltpu.sync_copy(pidx_hbm.at[pl.ds(wid * NCHUNK, NCHUNK)], pidx)
    nrows = BW * K // 128
    pltpu.sync_copy(nidx_hbm.at[pl.ds(wid * nrows, nrows)], nidx)

    iota = lax.iota(jnp.int32, 16)
    sems = (sem0, sem1)

    def copies(c, s):
        sem = sems[s]
        cps = [
            pltpu.make_async_copy(v_hbm.at[cidx.at[c]], v_buf.at[s], sem),
            pltpu.make_async_copy(u_hbm.at[pidx.at[c]], p_buf.at[s], sem),
        ]
        for j in range(5):
            cps.append(pltpu.make_async_copy(
                u_hbm.at[nidx.at[c * 5 + j]],
                n_buf.at[s].at[pl.ds(j * 128, 128)], sem))
        return cps

    def issue(c, s):
        for cp in copies(c, s):
            cp.start()

    def drain(c, s):
        for cp in copies(c, s):
            cp.wait()

    def compute(c, s):
        vb, pb, nb = v_buf.at[s], p_buf.at[s], n_buf.at[s]
        for gg in range(CB // 16):
            bvec = gg * 16 + iota
            nrow0 = bvec * K
            acc0 = (jnp.zeros((16,), jnp.float32),) * (K + 1)

            def dbody(d, accs):
                dvec = jnp.full((16,), d, jnp.int32)
                vv = plsc.load_gather(vb, [bvec, dvec])
                pv = plsc.load_gather(pb, [bvec, dvec])
                out = [accs[0] + vv * pv]
                for k in range(K):
                    nv = plsc.load_gather(nb, [nrow0 + k, dvec])
                    out.append(accs[k + 1] + vv * nv)
                return tuple(out)

            accs = lax.fori_loop(0, D, dbody, acc0)
            base = c * CB + gg * 16
            pos_sc[pl.ds(base, 16)] = accs[0]
            blvec = base + iota
            for k in range(K):
                plsc.store_scatter(
                    neg_sc, [blvec, jnp.full((16,), k, jnp.int32)],
                    accs[k + 1])

    issue(0, 0)
    issue(1, 1)

    def outer(i, carry):
        cc = i * 2
        for s in range(2):
            c = cc + s
            drain(c, s)
            compute(c, s)

            @pl.when(c + 2 < NCHUNK)
            def _():
                issue(c + 2, s)
        return carry

    lax.fori_loop(0, NCHUNK // 2, outer, 0)

    pltpu.sync_copy(pos_sc, pos_out.at[pl.ds(wid * BW, BW)])
    pltpu.sync_copy(neg_sc, neg_out.at[pl.ds(wid * BW, BW)])


def _loss_body(p_ref, n_ref, o_ref):
    def logsig(x):
        return jnp.minimum(x, 0.0) - jnp.log1p(jnp.exp(-jnp.abs(x)))

    tot = jnp.sum(logsig(p_ref[...])) + jnp.sum(logsig(-n_ref[...]))
    o_ref[0, 0] = -tot / jnp.float32(B)


_loss_call = pl.pallas_call(
    _loss_body,
    out_shape=jax.ShapeDtypeStruct((1, 1), jnp.float32),
    out_specs=pl.BlockSpec(memory_space=pltpu.MemorySpace.SMEM),
)


def kernel(center_words, pos_words, neg_words, v_embed, u_embed):
    c2 = center_words.reshape(NW * NCHUNK, CB)
    p2 = pos_words.reshape(NW * NCHUNK, CB)
    n2 = neg_words.reshape(B * K // 128, 128)
    pos_s, neg_s = _sc_scores(v_embed, u_embed, c2, p2, n2)
    loss = _loss_call(pos_s.reshape(128, 128), neg_s.reshape(B * K // 128, 128))
    return loss[0, 0]
